# Initial kernel scaffold; baseline (speedup 1.0000x reference)
import jax
import jax.numpy as jnp
from jax.experimental import pallas as pl


def kernel(x, Wr, W1, b1, W2, b2):
    raise NotImplementedError("write your pallas kernel here")


# speedup vs baseline: 1.9196x; 1.9196x over previous
import jax
import jax.numpy as jnp
from jax.experimental import pallas as pl
from jax.experimental.pallas import tpu as pltpu

_HIDDEN = 1024
_E = 16
_FFN = 4096
_NTOK = 256
_FCHUNK = 1024
_NF = _FFN // _FCHUNK


def _moe_body(x_ref, wr_ref, w1_ref, b1_ref, w2_ref, b2_ref, out_ref, wts_ref):
    e = pl.program_id(0)
    f = pl.program_id(1)

    lane = jax.lax.broadcasted_iota(jnp.int32, (_NTOK, _E), 1)

    @pl.when((e == 0) & (f == 0))
    def _router():
        logits = jax.lax.dot_general(
            x_ref[...], wr_ref[...], (((1,), (1,)), ((), ())),
            preferred_element_type=jnp.float32,
        )
        m1 = jnp.max(logits, axis=1, keepdims=True)
        i1 = jnp.min(jnp.where(logits == m1, lane, _E), axis=1, keepdims=True)
        masked = jnp.where(lane == i1, -jnp.inf, logits)
        m2 = jnp.max(masked, axis=1, keepdims=True)
        i2 = jnp.min(jnp.where(masked == m2, lane, _E), axis=1, keepdims=True)
        t = jnp.exp(m2 - m1)
        p1 = 1.0 / (1.0 + t)
        p2 = t / (1.0 + t)
        wts_ref[...] = jnp.where(lane == i1, p1, 0.0) + jnp.where(lane == i2, p2, 0.0)

    xb = x_ref[...].astype(jnp.bfloat16)
    h = jax.lax.dot_general(
        xb, w1_ref[0].astype(jnp.bfloat16), (((1,), (1,)), ((), ())),
        preferred_element_type=jnp.float32,
    )
    h = h + b1_ref[0]
    a = 0.5 * h * (1.0 + jax.lax.erf(h * 0.7071067811865476))
    o = jax.lax.dot_general(
        a.astype(jnp.bfloat16), w2_ref[0].astype(jnp.bfloat16),
        (((1,), (1,)), ((), ())),
        preferred_element_type=jnp.float32,
    )
    o = jnp.where(f == 0, o + b2_ref[0], o)
    wcol = jnp.sum(wts_ref[...] * (lane == e).astype(jnp.float32),
                   axis=1, keepdims=True)
    contrib = wcol * o

    @pl.when((e == 0) & (f == 0))
    def _init():
        out_ref[...] = contrib

    @pl.when(~((e == 0) & (f == 0)))
    def _acc():
        out_ref[...] += contrib


def kernel(x, Wr, W1, b1, W2, b2):
    B, S, D = x.shape
    xf = x.reshape(B * S, D)
    b1r = b1.reshape(_E * _NF, 1, _FCHUNK)
    b2r = b2.reshape(_E, 1, _HIDDEN)
    out = pl.pallas_call(
        _moe_body,
        grid=(_E, _NF),
        in_specs=[
            pl.BlockSpec((_NTOK, _HIDDEN), lambda e, f: (0, 0)),
            pl.BlockSpec((_E, _HIDDEN), lambda e, f: (0, 0)),
            pl.BlockSpec((1, _FCHUNK, _HIDDEN), lambda e, f: (e, f, 0)),
            pl.BlockSpec((1, 1, _FCHUNK), lambda e, f: (e * _NF + f, 0, 0)),
            pl.BlockSpec((1, _HIDDEN, _FCHUNK), lambda e, f: (e, 0, f)),
            pl.BlockSpec((1, 1, _HIDDEN), lambda e, f: (e, 0, 0)),
        ],
        out_specs=pl.BlockSpec((_NTOK, _HIDDEN), lambda e, f: (0, 0)),
        out_shape=jax.ShapeDtypeStruct((_NTOK, _HIDDEN), jnp.float32),
        scratch_shapes=[pltpu.VMEM((_NTOK, _E), jnp.float32)],
        compiler_params=pltpu.CompilerParams(
            dimension_semantics=("arbitrary", "arbitrary"),
        ),
    )(xf, Wr, W1, b1r, W2, b2r)
    return out.reshape(B, S, D)


# speedup vs baseline: 2.0079x; 1.0460x over previous
import jax
import jax.numpy as jnp
from jax.experimental import pallas as pl
from jax.experimental.pallas import tpu as pltpu

_HIDDEN = 1024
_E = 16
_FFN = 4096
_NTOK = 256
_FCHUNK = 2048
_NF = _FFN // _FCHUNK


def _moe_body(x_ref, wr_ref, w1_ref, b1_ref, w2_ref, b2_ref, out_ref, wts_ref):
    e = pl.program_id(0)
    f = pl.program_id(1)

    lane = jax.lax.broadcasted_iota(jnp.int32, (_NTOK, _E), 1)

    @pl.when((e == 0) & (f == 0))
    def _router():
        logits = jax.lax.dot_general(
            x_ref[...], wr_ref[...], (((1,), (1,)), ((), ())),
            preferred_element_type=jnp.float32,
        )
        m1 = jnp.max(logits, axis=1, keepdims=True)
        i1 = jnp.min(jnp.where(logits == m1, lane, _E), axis=1, keepdims=True)
        masked = jnp.where(lane == i1, -jnp.inf, logits)
        m2 = jnp.max(masked, axis=1, keepdims=True)
        i2 = jnp.min(jnp.where(masked == m2, lane, _E), axis=1, keepdims=True)
        t = jnp.exp(m2 - m1)
        p1 = 1.0 / (1.0 + t)
        p2 = t / (1.0 + t)
        wts_ref[...] = jnp.where(lane == i1, p1, 0.0) + jnp.where(lane == i2, p2, 0.0)

    xb = x_ref[...].astype(jnp.bfloat16)
    h = jax.lax.dot_general(
        xb, w1_ref[0].astype(jnp.bfloat16), (((1,), (1,)), ((), ())),
        preferred_element_type=jnp.float32,
    )
    h = h + b1_ref[0]
    a = 0.5 * h * (1.0 + jax.lax.erf(h * 0.7071067811865476))
    o = jax.lax.dot_general(
        a.astype(jnp.bfloat16), w2_ref[0].astype(jnp.bfloat16),
        (((1,), (1,)), ((), ())),
        preferred_element_type=jnp.float32,
    )
    o = jnp.where(f == 0, o + b2_ref[0], o)
    wcol = jnp.sum(wts_ref[...] * (lane == e).astype(jnp.float32),
                   axis=1, keepdims=True)
    contrib = wcol * o

    @pl.when((e == 0) & (f == 0))
    def _init():
        out_ref[...] = contrib

    @pl.when(~((e == 0) & (f == 0)))
    def _acc():
        out_ref[...] += contrib


def kernel(x, Wr, W1, b1, W2, b2):
    B, S, D = x.shape
    xf = x.reshape(B * S, D)
    b1r = b1.reshape(_E * _NF, 1, _FCHUNK)
    b2r = b2.reshape(_E, 1, _HIDDEN)
    out = pl.pallas_call(
        _moe_body,
        grid=(_E, _NF),
        in_specs=[
            pl.BlockSpec((_NTOK, _HIDDEN), lambda e, f: (0, 0)),
            pl.BlockSpec((_E, _HIDDEN), lambda e, f: (0, 0)),
            pl.BlockSpec((1, _FCHUNK, _HIDDEN), lambda e, f: (e, f, 0)),
            pl.BlockSpec((1, 1, _FCHUNK), lambda e, f: (e * _NF + f, 0, 0)),
            pl.BlockSpec((1, _HIDDEN, _FCHUNK), lambda e, f: (e, 0, f)),
            pl.BlockSpec((1, 1, _HIDDEN), lambda e, f: (e, 0, 0)),
        ],
        out_specs=pl.BlockSpec((_NTOK, _HIDDEN), lambda e, f: (0, 0)),
        out_shape=jax.ShapeDtypeStruct((_NTOK, _HIDDEN), jnp.float32),
        scratch_shapes=[pltpu.VMEM((_NTOK, _E), jnp.float32)],
        compiler_params=pltpu.CompilerParams(
            dimension_semantics=("arbitrary", "arbitrary"),
        ),
    )(xf, Wr, W1, b1r, W2, b2r)
    return out.reshape(B, S, D)
